# baseline (speedup 1.0000x reference)
import functools

import jax
import jax.numpy as jnp
from jax import lax
from jax.experimental import pallas as pl
from jax.experimental.pallas import tpu as pltpu
from jax.experimental.pallas import tpu_sc as plsc

BATCH = 16384
EMBED_DIM = 64
NUM_WORKERS = 32
B_PER_W = BATCH // NUM_WORKERS
CHUNK = 128
NCHUNK = B_PER_W // CHUNK
GROUPS = B_PER_W // 16


def _gmf_kernel(user_idx_hbm, item_idx_hbm, user_table, item_table, wb_hbm,
                out_hbm, idx_u, idx_i, u_rows, i_rows, w_v, out_v, sem):
    wid = lax.axis_index("s") * 2 + lax.axis_index("c")
    base = wid * B_PER_W

    pltpu.sync_copy(user_idx_hbm.at[pl.ds(wid * NCHUNK, NCHUNK)], idx_u)
    pltpu.sync_copy(item_idx_hbm.at[pl.ds(wid * NCHUNK, NCHUNK)], idx_i)
    pltpu.sync_copy(wb_hbm, w_v)

    copies = []
    for c in range(NCHUNK):
        copies.append(pltpu.async_copy(
            user_table.at[idx_u.at[c]],
            u_rows.at[pl.ds(c * CHUNK, CHUNK)], sem))
        copies.append(pltpu.async_copy(
            item_table.at[idx_i.at[c]],
            i_rows.at[pl.ds(c * CHUNK, CHUNK)], sem))
    for cp in copies:
        cp.wait()

    w_chunks = [w_v[0, pl.ds(c * 16, 16)] for c in range(5)]
    w_s = [w_chunks[d // 16][d % 16] for d in range(EMBED_DIM)]
    b_s = w_chunks[4][0]
    lane = lax.broadcasted_iota(jnp.int32, (16,), 0)

    def group_body(g, carry):
        row_idx = g * 16 + lane
        acc = jnp.zeros((16,), jnp.float32)
        for d in range(EMBED_DIM):
            d_idx = jnp.full((16,), d, jnp.int32)
            u_col = plsc.load_gather(u_rows, [row_idx, d_idx])
            i_col = plsc.load_gather(i_rows, [row_idx, d_idx])
            acc = acc + u_col * i_col * w_s[d]
        x = acc + b_s
        y = 1.0 / (1.0 + jnp.exp(-x))
        out_v[pl.ds(g * 16, 16)] = y
        return carry

    lax.fori_loop(0, GROUPS, group_body, 0)

    pltpu.sync_copy(out_v, out_hbm.at[pl.ds(base, B_PER_W)])


@jax.jit
def _gmf(user_indices, item_indices, user_table, item_table, W, b):
    mesh = plsc.VectorSubcoreMesh(core_axis_name="c", subcore_axis_name="s")
    kern = functools.partial(
        pl.kernel,
        mesh=mesh,
        out_type=jax.ShapeDtypeStruct((BATCH,), jnp.float32),
        scratch_types=[
            pltpu.VMEM((NCHUNK, CHUNK), jnp.int32),
            pltpu.VMEM((NCHUNK, CHUNK), jnp.int32),
            pltpu.VMEM((B_PER_W, EMBED_DIM), jnp.float32),
            pltpu.VMEM((B_PER_W, EMBED_DIM), jnp.float32),
            pltpu.VMEM((1, EMBED_DIM + 16), jnp.float32),
            pltpu.VMEM((B_PER_W,), jnp.float32),
            pltpu.SemaphoreType.DMA,
        ],
        compiler_params=pltpu.CompilerParams(
            needs_layout_passes=False, use_tc_tiling_on_sc=False),
    )(_gmf_kernel)
    uidx = user_indices.astype(jnp.int32).reshape(
        NUM_WORKERS * NCHUNK, CHUNK)
    iidx = item_indices.astype(jnp.int32).reshape(
        NUM_WORKERS * NCHUNK, CHUNK)
    wb = jnp.concatenate(
        [W.astype(jnp.float32),
         jnp.pad(b.astype(jnp.float32), (0, 15)).reshape(1, 16)], axis=1)
    return kern(uidx, iidx, user_table, item_table, wb)


def kernel(user_indices, item_indices, user_table, item_table, W, b):
    return _gmf(user_indices, item_indices, user_table, item_table, W, b)


# speedup vs baseline: 2.2635x; 2.2635x over previous
import functools

import jax
import jax.numpy as jnp
from jax import lax
from jax.experimental import pallas as pl
from jax.experimental.pallas import tpu as pltpu
from jax.experimental.pallas import tpu_sc as plsc

BATCH = 16384
EMBED_DIM = 64
NUM_WORKERS = 32
B_PER_W = BATCH // NUM_WORKERS
GROUPS = B_PER_W // 16
LANE = 128


def _gmf_kernel(user_idx_hbm, item_idx_hbm, user_t, item_t, wb_hbm,
                out_hbm, idx_uv, idx_iv, ub0, ub1, ib0, ib1,
                u_cols, i_cols, w_v, out_v, su0, su1, si0, si1):
    wid = lax.axis_index("s") * 2 + lax.axis_index("c")
    base = wid * B_PER_W

    pltpu.sync_copy(user_idx_hbm.at[pl.ds(base, B_PER_W)], idx_uv)
    pltpu.sync_copy(item_idx_hbm.at[pl.ds(base, B_PER_W)], idx_iv)
    pltpu.sync_copy(wb_hbm, w_v)

    ubufs = (ub0, ub1)
    ibufs = (ib0, ib1)
    usems = (su0, su1)
    isems = (si0, si1)

    def fire(u, i, slot):
        ucol = pl.multiple_of((u >> 7) << 7, LANE)
        icol = pl.multiple_of((i >> 7) << 7, LANE)
        pltpu.async_copy(
            user_t.at[pl.ds(0, EMBED_DIM), pl.ds(ucol, LANE)],
            ubufs[slot], usems[slot])
        pltpu.async_copy(
            item_t.at[pl.ds(0, EMBED_DIM), pl.ds(icol, LANE)],
            ibufs[slot], isems[slot])

    lane = lax.broadcasted_iota(jnp.int32, (16,), 0)

    def extract(b, u, i, slot):
        pltpu.make_async_copy(
            user_t.at[pl.ds(0, EMBED_DIM), pl.ds(0, LANE)],
            ubufs[slot], usems[slot]).wait()
        pltpu.make_async_copy(
            item_t.at[pl.ds(0, EMBED_DIM), pl.ds(0, LANE)],
            ibufs[slot], isems[slot]).wait()
        ul = jnp.full((16,), u & 127, jnp.int32)
        il = jnp.full((16,), i & 127, jnp.int32)
        for c in range(4):
            d_idx = c * 16 + lane
            uv = plsc.load_gather(ubufs[slot], [d_idx, ul])
            iv = plsc.load_gather(ibufs[slot], [d_idx, il])
            plsc.store_scatter(u_cols, [d_idx * B_PER_W + b], uv)
            plsc.store_scatter(i_cols, [d_idx * B_PER_W + b], iv)

    u0 = idx_uv[pl.ds(0, 16)]
    i0 = idx_iv[pl.ds(0, 16)]
    fire(u0[0], i0[0], 0)

    def pipe_body(g, carry):
        off = g * 16
        u16 = idx_uv[pl.ds(off, 16)]
        i16 = idx_iv[pl.ds(off, 16)]
        offn = jnp.minimum(g + 1, GROUPS - 1) * 16
        u16n = idx_uv[pl.ds(offn, 16)]
        i16n = idx_iv[pl.ds(offn, 16)]
        for k in range(16):
            b = off + k
            un = u16[k + 1] if k < 15 else u16n[0]
            inx = i16[k + 1] if k < 15 else i16n[0]

            @pl.when(b + 1 < B_PER_W)
            def _():
                fire(un, inx, (k + 1) & 1)

            extract(b, u16[k], i16[k], k & 1)
        return carry

    lax.fori_loop(0, GROUPS, pipe_body, 0)

    w_chunks = [w_v[pl.ds(c * 16, 16)] for c in range(5)]
    w_s = [w_chunks[d // 16][d % 16] for d in range(EMBED_DIM)]
    b_s = w_chunks[4][0]

    def group_body(g, carry):
        off = g * 16
        acc = jnp.zeros((16,), jnp.float32)
        for d in range(EMBED_DIM):
            u_v = u_cols[pl.ds(d * B_PER_W + off, 16)]
            i_v = i_cols[pl.ds(d * B_PER_W + off, 16)]
            acc = acc + u_v * i_v * w_s[d]
        x = acc + b_s
        y = 1.0 / (1.0 + jnp.exp(-x))
        out_v[pl.ds(off, 16)] = y
        return carry

    lax.fori_loop(0, GROUPS, group_body, 0)

    pltpu.sync_copy(out_v, out_hbm.at[pl.ds(base, B_PER_W)])


@jax.jit
def _gmf(user_indices, item_indices, user_table, item_table, W, b):
    mesh = plsc.VectorSubcoreMesh(core_axis_name="c", subcore_axis_name="s")
    kern = functools.partial(
        pl.kernel,
        mesh=mesh,
        out_type=jax.ShapeDtypeStruct((BATCH,), jnp.float32),
        scratch_types=[
            pltpu.VMEM((B_PER_W,), jnp.int32),
            pltpu.VMEM((B_PER_W,), jnp.int32),
            pltpu.VMEM((EMBED_DIM, LANE), jnp.float32),
            pltpu.VMEM((EMBED_DIM, LANE), jnp.float32),
            pltpu.VMEM((EMBED_DIM, LANE), jnp.float32),
            pltpu.VMEM((EMBED_DIM, LANE), jnp.float32),
            pltpu.VMEM((EMBED_DIM * B_PER_W,), jnp.float32),
            pltpu.VMEM((EMBED_DIM * B_PER_W,), jnp.float32),
            pltpu.VMEM((EMBED_DIM + 16,), jnp.float32),
            pltpu.VMEM((B_PER_W,), jnp.float32),
            pltpu.SemaphoreType.DMA,
            pltpu.SemaphoreType.DMA,
            pltpu.SemaphoreType.DMA,
            pltpu.SemaphoreType.DMA,
        ],
        compiler_params=pltpu.CompilerParams(needs_layout_passes=False),
    )(_gmf_kernel)
    uidx = user_indices.astype(jnp.int32)
    iidx = item_indices.astype(jnp.int32)
    wb = jnp.concatenate(
        [W.astype(jnp.float32).reshape(EMBED_DIM),
         jnp.pad(b.astype(jnp.float32), (0, 15))])
    return kern(uidx, iidx, user_table.T, item_table.T, wb)


def kernel(user_indices, item_indices, user_table, item_table, W, b):
    return _gmf(user_indices, item_indices, user_table, item_table, W, b)
